# baseline (speedup 1.0000x reference)
import functools
import math

import jax
import jax.numpy as jnp
import numpy as np
from jax import lax
from jax.experimental import pallas as pl
from jax.experimental.pallas import tpu as pltpu
from jax.experimental.pallas import tpu_sc as plsc

D = 128
SEQ = 200
UNIT = 100
NBUF = 6
LEAD = 4
SCALE = math.sqrt(float(D))


def _positional_encoding(length, depth):
    half = depth // 2
    positions = np.arange(length)[:, None].astype(np.float32)
    depths = np.arange(half)[None, :].astype(np.float32) / float(half)
    angle_rates = 1.0 / (10000.0 ** depths)
    angle_rads = positions * angle_rates
    return np.concatenate([np.sin(angle_rads), np.cos(angle_rads)], axis=-1)


_PE = jnp.asarray(_positional_encoding(2048, D)[:SEQ], dtype=jnp.float32)


@functools.cache
def _make_kernel(n_tokens):
    info = plsc.get_sparse_core_info()
    nc, ns = info.num_cores, info.num_subcores
    nw = nc * ns
    upw = n_tokens // (nw * UNIT)
    mesh = plsc.VectorSubcoreMesh(core_axis_name="c", subcore_axis_name="s")
    n_groups = (upw - LEAD) // NBUF
    tail = upw - n_groups * NBUF

    @functools.partial(
        pl.kernel,
        out_type=jax.ShapeDtypeStruct((n_tokens // UNIT, UNIT, D),
                                      jnp.float32),
        mesh=mesh,
        scratch_types=[
            pltpu.VMEM((upw, UNIT), jnp.int32),
            pltpu.VMEM((SEQ, D), jnp.float32),
        ] + [pltpu.VMEM((UNIT, D), jnp.float32)] * NBUF
          + [pltpu.SemaphoreType.DMA] * (2 * NBUF),
    )
    def k(x_hbm, table_hbm, pe_hbm, out_hbm, idx_v, pe_v,
          r0, r1, r2, r3, r4, r5,
          g0, g1, g2, g3, g4, g5, o0, o1, o2, o3, o4, o5):
        rows = (r0, r1, r2, r3, r4, r5)
        gsem = (g0, g1, g2, g3, g4, g5)
        osem = (o0, o1, o2, o3, o4, o5)
        wid = lax.axis_index("s") * nc + lax.axis_index("c")
        pltpu.sync_copy(x_hbm.at[pl.ds(wid * upw, upw)], idx_v)
        pltpu.sync_copy(pe_hbm, pe_v)

        def gather(u, b):
            return pltpu.make_async_copy(
                table_hbm.at[idx_v.at[u]], rows[b], gsem[b])

        def out_cp(u, b):
            return pltpu.make_async_copy(
                rows[b], out_hbm.at[wid * upw + u], osem[b])

        def compute(b, po):
            buf = rows[b]
            ng = D // 16

            def row_body(t, c):
                embs = [buf[t, pl.ds(g * 16, 16)] for g in range(ng)]
                pes = [pe_v[t + po, pl.ds(g * 16, 16)] for g in range(ng)]
                for g in range(ng):
                    buf[t, pl.ds(g * 16, 16)] = embs[g] * SCALE + pes[g]
                return c

            lax.fori_loop(0, UNIT, row_body, 0)

        for j in range(LEAD):
            gather(j, j).start()

        def group(p, carry):
            for j in range(NBUF):
                u = NBUF * p + j
                nb = (j + LEAD) % NBUF
                gather(u, j).wait()
                compute(j, (j & 1) * UNIT)
                out_cp(u, j).start()

                @pl.when(u >= NBUF - LEAD)
                def _():
                    out_cp(u - (NBUF - LEAD), nb).wait()

                gather(u + LEAD, nb).start()
            return carry

        lax.fori_loop(0, n_groups, group, 0)

        for j in range(tail):
            u = n_groups * NBUF + j
            b = u % NBUF
            gather(u, b).wait()
            compute(b, (j & 1) * UNIT)
            out_cp(u, b).start()
        for j in range(NBUF):
            u = upw - NBUF + j
            out_cp(u, u % NBUF).wait()

    return k


def kernel(x, table):
    n_batch, seq = x.shape
    n_tokens = n_batch * seq
    out = _make_kernel(n_tokens)(
        x.reshape(n_tokens // UNIT, UNIT), table, _PE)
    return out.reshape(n_batch, seq, D)


# speedup vs baseline: 1.3630x; 1.3630x over previous
import functools
import math

import jax
import jax.numpy as jnp
import numpy as np
from jax import lax
from jax.experimental import pallas as pl
from jax.experimental.pallas import tpu as pltpu
from jax.experimental.pallas import tpu_sc as plsc

D = 128
SEQ = 200
UNIT = 160
NBUF = 4
SCALE = math.sqrt(float(D))


def _positional_encoding(length, depth):
    half = depth // 2
    positions = np.arange(length)[:, None].astype(np.float32)
    depths = np.arange(half)[None, :].astype(np.float32) / float(half)
    angle_rates = 1.0 / (10000.0 ** depths)
    angle_rads = positions * angle_rates
    return np.concatenate([np.sin(angle_rads), np.cos(angle_rads)], axis=-1)


_PE = jnp.asarray(_positional_encoding(2048, D)[:SEQ], dtype=jnp.float32)


@functools.cache
def _make_kernel(n_tokens):
    info = plsc.get_sparse_core_info()
    nc, ns = info.num_cores, info.num_subcores
    nw = nc * ns
    upw = n_tokens // (nw * UNIT)
    n_pairs = upw // 4
    mesh = plsc.VectorSubcoreMesh(core_axis_name="c", subcore_axis_name="s")

    @functools.partial(
        pl.kernel,
        out_type=jax.ShapeDtypeStruct((n_tokens // UNIT, UNIT, D),
                                      jnp.float32),
        mesh=mesh,
        scratch_types=[
            pltpu.VMEM((upw * UNIT,), jnp.int32),
            pltpu.VMEM((SEQ, D), jnp.float32),
        ] + [pltpu.VMEM((UNIT, D), jnp.float32)] * NBUF
          + [pltpu.SemaphoreType.DMA] * (2 * NBUF),
    )
    def k(x_hbm, table_hbm, pe_hbm, out_hbm, idx_v, pe_v,
          r0, r1, r2, r3, g0, g1, g2, g3, o0, o1, o2, o3):
        rows = (r0, r1, r2, r3)
        gsem = (g0, g1, g2, g3)
        osem = (o0, o1, o2, o3)
        wid = lax.axis_index("s") * nc + lax.axis_index("c")
        pltpu.sync_copy(x_hbm.at[pl.ds(wid * upw * UNIT, upw * UNIT)],
                        idx_v)
        pltpu.sync_copy(pe_hbm, pe_v)

        def gather(u, b):
            return pltpu.make_async_copy(
                table_hbm.at[idx_v.at[pl.ds(u * UNIT, UNIT)]], rows[b],
                gsem[b])

        def out_cp(u, b):
            return pltpu.make_async_copy(
                rows[b], out_hbm.at[wid * upw + u], osem[b])

        def compute(b, u):
            buf = rows[b]
            ng = D // 16
            po = lax.rem(u * UNIT, SEQ)

            def row_body(t, c):
                embs = [buf[t, pl.ds(g * 16, 16)] for g in range(ng)]
                pes = [pe_v[c, pl.ds(g * 16, 16)] for g in range(ng)]
                for g in range(ng):
                    buf[t, pl.ds(g * 16, 16)] = embs[g] * SCALE + pes[g]
                return lax.select(c + 1 == SEQ, 0, c + 1)

            lax.fori_loop(0, UNIT, row_body, po)

        gather(0, 0).start()
        gather(1, 1).start()

        def pair(p, carry):
            uA = 4 * p
            uB = 4 * p + 2
            for j in range(2):
                gather(uA + j, j).wait()
                compute(j, uA + j)
            for j in range(2):
                out_cp(uA + j, j).start()

            @pl.when(p >= 1)
            def _():
                for j in range(2):
                    out_cp(uA - 2 + j, 2 + j).wait()

            for j in range(2):
                gather(uB + j, 2 + j).start()
            for j in range(2):
                gather(uB + j, 2 + j).wait()
                compute(2 + j, uB + j)
            for j in range(2):
                out_cp(uB + j, 2 + j).start()
            for j in range(2):
                out_cp(uA + j, j).wait()

            @pl.when(p + 1 < n_pairs)
            def _():
                for j in range(2):
                    gather(uB + 2 + j, j).start()

            return carry

        lax.fori_loop(0, n_pairs, pair, 0)
        for j in range(2):
            out_cp(upw - 2 + j, 2 + j).wait()

    return k


def kernel(x, table):
    n_batch, seq = x.shape
    n_tokens = n_batch * seq
    out = _make_kernel(n_tokens)(x.reshape(-1), table, _PE)
    return out.reshape(n_batch, seq, D)
